# baseline (speedup 1.0000x reference)
import functools

import jax
import jax.numpy as jnp
from jax import lax
from jax.experimental import pallas as pl
from jax.experimental.pallas import tpu as pltpu
from jax.experimental.pallas import tpu_sc as plsc


def _make_gather(V, D, B):
    info = plsc.get_sparse_core_info()
    NC, NS = info.num_cores, info.num_subcores
    NW = NC * NS
    assert B % (8 * NW) == 0
    b_per_w = B // NW
    mesh = plsc.VectorSubcoreMesh(core_axis_name="c", subcore_axis_name="s")

    @functools.partial(
        pl.kernel,
        mesh=mesh,
        out_type=jax.ShapeDtypeStruct((B, D), jnp.float32),
        scratch_types=[
            pltpu.VMEM((b_per_w,), jnp.int32),
            pltpu.VMEM((2, b_per_w // 4, D), jnp.float32),
            pltpu.SemaphoreType.DMA,
            pltpu.SemaphoreType.DMA,
            pltpu.SemaphoreType.DMA,
        ],
    )
    def gather_kernel(table_hbm, idx_hbm, out_hbm, idx_v, rows_v, g0, g1, wsem):
        wid = lax.axis_index("s") * NC + lax.axis_index("c")
        base = wid * b_per_w
        chunk = b_per_w // 4
        gsems = (g0, g1)
        pltpu.sync_copy(idx_hbm.at[pl.ds(base, b_per_w)], idx_v)
        gathers = [None, None]
        writes = [None, None]
        for c in range(4):
            buf = c % 2
            if writes[buf] is not None:
                writes[buf].wait()
            gathers[buf] = pltpu.async_copy(
                table_hbm.at[idx_v.at[pl.ds(c * chunk, chunk)]],
                rows_v.at[buf],
                gsems[buf],
            )
            if c >= 1:
                pbuf = (c - 1) % 2
                gathers[pbuf].wait()
                writes[pbuf] = pltpu.async_copy(
                    rows_v.at[pbuf],
                    out_hbm.at[pl.ds(base + (c - 1) * chunk, chunk)],
                    wsem,
                )
        gathers[1].wait()
        writes[1] = pltpu.async_copy(
            rows_v.at[1], out_hbm.at[pl.ds(base + 3 * chunk, chunk)], wsem
        )
        writes[0].wait()
        writes[1].wait()

    return gather_kernel


def kernel(tool_ids, tool_semantic_embeddings):
    V, D = tool_semantic_embeddings.shape
    B = tool_ids.shape[0]
    idx = tool_ids.astype(jnp.int32)
    return _make_gather(V, D, B)(tool_semantic_embeddings, idx)


# speedup vs baseline: 1.0423x; 1.0423x over previous
import functools

import jax
import jax.numpy as jnp
from jax import lax
from jax.experimental import pallas as pl
from jax.experimental.pallas import tpu as pltpu
from jax.experimental.pallas import tpu_sc as plsc


def _make_gather(V, D, B):
    info = plsc.get_sparse_core_info()
    NC, NS = info.num_cores, info.num_subcores
    NW = NC * NS
    assert B % (8 * NW) == 0
    b_per_w = B // NW
    mesh = plsc.VectorSubcoreMesh(core_axis_name="c", subcore_axis_name="s")

    @functools.partial(
        pl.kernel,
        mesh=mesh,
        out_type=jax.ShapeDtypeStruct((B, D), jnp.float32),
        scratch_types=[
            pltpu.VMEM((b_per_w,), jnp.int32),
            pltpu.VMEM((b_per_w, D), jnp.float32),
            pltpu.SemaphoreType.DMA,
        ],
    )
    def gather_kernel(table_hbm, idx_hbm, out_hbm, idx_v, rows_v, sem):
        wid = lax.axis_index("s") * NC + lax.axis_index("c")
        base = wid * b_per_w
        pltpu.sync_copy(idx_hbm.at[pl.ds(base, b_per_w)], idx_v)
        pltpu.async_copy(table_hbm.at[idx_v], rows_v, sem).wait()
        pltpu.sync_copy(rows_v, out_hbm.at[pl.ds(base, b_per_w)])

    return gather_kernel


def kernel(tool_ids, tool_semantic_embeddings):
    V, D = tool_semantic_embeddings.shape
    B = tool_ids.shape[0]
    idx = tool_ids.astype(jnp.int32)
    return _make_gather(V, D, B)(tool_semantic_embeddings, idx)
